# baseline (speedup 1.0000x reference)
import functools

import jax
import jax.numpy as jnp
from jax import lax
from jax.experimental import pallas as pl
from jax.experimental.pallas import tpu as pltpu
from jax.experimental.pallas import tpu_sc as plsc

VOCAB = 1000000
B = 16384
EMB = 64
PAIR = 2 * EMB
INPUT_DIM = 2 * EMB
HID = 128
OUT = 64

NC = 2
NS = 16
NW = NC * NS
B_PER_W = B // NW
CHUNK = 128
NCHUNK = B_PER_W // CHUNK


def _sc_gather_body(tab_t_hbm, tab_l_hbm, idx_t_hbm, idx_l_hbm,
                    out_t_hbm, out_l_hbm,
                    idx_t_v, idx_l_v, rows_v, sem):
  wid = lax.axis_index("s") * NC + lax.axis_index("c")
  base_chunk = wid * NCHUNK
  base_row = wid * B_PER_W

  pltpu.sync_copy(idx_t_hbm.at[pl.ds(base_chunk, NCHUNK)], idx_t_v)
  pltpu.sync_copy(idx_l_hbm.at[pl.ds(base_chunk, NCHUNK)], idx_l_v)

  copies = []
  for j in range(NCHUNK):
    copies.append(pltpu.async_copy(
        tab_t_hbm.at[idx_t_v.at[j]], rows_v.at[pl.ds(j * CHUNK, CHUNK)],
        sem))
  for c in copies:
    c.wait()
  pltpu.sync_copy(rows_v, out_t_hbm.at[pl.ds(base_row, B_PER_W)])

  copies = []
  for j in range(NCHUNK):
    copies.append(pltpu.async_copy(
        tab_l_hbm.at[idx_l_v.at[j]], rows_v.at[pl.ds(j * CHUNK, CHUNK)],
        sem))
  for c in copies:
    c.wait()
  pltpu.sync_copy(rows_v, out_l_hbm.at[pl.ds(base_row, B_PER_W)])


@functools.cache
def _sc_gather():
  return pl.kernel(
      _sc_gather_body,
      out_type=(
          jax.ShapeDtypeStruct((B, PAIR), jnp.float32),
          jax.ShapeDtypeStruct((B, PAIR), jnp.float32),
      ),
      mesh=plsc.VectorSubcoreMesh(core_axis_name="c", subcore_axis_name="s"),
      scratch_types=[
          pltpu.VMEM((NCHUNK, CHUNK), jnp.int32),
          pltpu.VMEM((NCHUNK, CHUNK), jnp.int32),
          pltpu.VMEM((B_PER_W, PAIR), jnp.float32),
          pltpu.SemaphoreType.DMA,
      ],
  )


BT = 2048


def _mlp_body(pt_ref, pl_ref, part_ref, parl_ref, ln1w_ref, ln1b_ref,
              w1t_ref, b1_ref, w2t_ref, b2_ref, ln2w_ref, ln2b_ref, out_ref):
  pt = pt_ref[...]
  plc = pl_ref[...]
  et = jnp.where(part_ref[...] > 0.5, pt[:, EMB:], pt[:, :EMB])
  el = jnp.where(parl_ref[...] > 0.5, plc[:, EMB:], plc[:, :EMB])
  x = jnp.concatenate([et, el], axis=1)
  mu = jnp.mean(x, axis=1, keepdims=True)
  xc = x - mu
  var = jnp.mean(xc * xc, axis=1, keepdims=True)
  h = xc * jax.lax.rsqrt(var + 1e-5) * ln1w_ref[...] + ln1b_ref[...]
  h = jnp.dot(h, w1t_ref[...], preferred_element_type=jnp.float32)
  h = jnp.maximum(h + b1_ref[...], 0.0)
  y = jnp.dot(h, w2t_ref[...], preferred_element_type=jnp.float32)
  y = y + b2_ref[...]
  mu2 = jnp.mean(y, axis=1, keepdims=True)
  yc = y - mu2
  var2 = jnp.mean(yc * yc, axis=1, keepdims=True)
  out_ref[...] = yc * jax.lax.rsqrt(var2 + 1e-5) * ln2w_ref[...] + ln2b_ref[...]


def _mlp(pt, plc, part, parl, ln1w, ln1b, w1t, b1, w2t, b2, ln2w, ln2b):
  full = lambda shape: pl.BlockSpec(shape, lambda i: (0, 0))
  return pl.pallas_call(
      _mlp_body,
      grid=(B // BT,),
      in_specs=[
          pl.BlockSpec((BT, PAIR), lambda i: (i, 0)),
          pl.BlockSpec((BT, PAIR), lambda i: (i, 0)),
          pl.BlockSpec((BT, 1), lambda i: (i, 0)),
          pl.BlockSpec((BT, 1), lambda i: (i, 0)),
          full((1, INPUT_DIM)), full((1, INPUT_DIM)),
          full((INPUT_DIM, HID)), full((1, HID)),
          full((HID, OUT)), full((1, OUT)),
          full((1, OUT)), full((1, OUT)),
      ],
      out_specs=pl.BlockSpec((BT, OUT), lambda i: (i, 0)),
      out_shape=jax.ShapeDtypeStruct((B, OUT), jnp.float32),
  )(pt, plc, part, parl, ln1w, ln1b, w1t, b1, w2t, b2, ln2w, ln2b)


def kernel(Type, Location, emb_type, emb_loc, ln1_w, ln1_b, w1, b1, w2, b2,
           ln2_w, ln2_b):
  it = Type.astype(jnp.int32)
  il = Location.astype(jnp.int32)
  idx_t = jnp.reshape(it >> 1, (B // CHUNK, CHUNK))
  idx_l = jnp.reshape(il >> 1, (B // CHUNK, CHUNK))
  part = (it & 1).astype(jnp.float32).reshape(B, 1)
  parl = (il & 1).astype(jnp.float32).reshape(B, 1)
  tab_t = jnp.reshape(emb_type, (VOCAB // 2, PAIR))
  tab_l = jnp.reshape(emb_loc, (VOCAB // 2, PAIR))
  pt, plc = _sc_gather()(tab_t, tab_l, idx_t, idx_l)
  return _mlp(
      pt, plc, part, parl,
      ln1_w.reshape(1, INPUT_DIM), ln1_b.reshape(1, INPUT_DIM),
      w1.T, b1.reshape(1, HID),
      w2.T, b2.reshape(1, OUT),
      ln2_w.reshape(1, OUT), ln2_b.reshape(1, OUT),
  )


# speedup vs baseline: 1.6273x; 1.6273x over previous
import functools

import jax
import jax.numpy as jnp
from jax import lax
from jax.experimental import pallas as pl
from jax.experimental.pallas import tpu as pltpu
from jax.experimental.pallas import tpu_sc as plsc

VOCAB = 1000000
B = 16384
EMB = 64
PAIR = 2 * EMB
INPUT_DIM = 2 * EMB
HID = 128
OUT = 64

NC = 2
NS = 16
NW = NC * NS
B_PER_W = B // NW
CHUNK = 128
NCHUNK = B_PER_W // CHUNK

TCOLS = 2048
HCOLS = TCOLS // 2
TGRID = (VOCAB + TCOLS - 1) // TCOLS
NPAIR = TGRID * HCOLS


def _pack_pairs(x):
  xt = x.T
  return jnp.concatenate([xt[:HCOLS], xt[HCOLS:]], axis=1)


def _transpose_body(at_ref, al_ref, ot_ref, ol_ref):
  ot_ref[...] = _pack_pairs(at_ref[...])
  ol_ref[...] = _pack_pairs(al_ref[...])


def _transpose(tabT_t, tabT_l):
  return pl.pallas_call(
      _transpose_body,
      grid=(TGRID,),
      in_specs=[
          pl.BlockSpec((EMB, TCOLS), lambda i: (0, i)),
          pl.BlockSpec((EMB, TCOLS), lambda i: (0, i)),
      ],
      out_specs=[
          pl.BlockSpec((HCOLS, PAIR), lambda i: (i, 0)),
          pl.BlockSpec((HCOLS, PAIR), lambda i: (i, 0)),
      ],
      out_shape=[
          jax.ShapeDtypeStruct((NPAIR, PAIR), jnp.float32),
          jax.ShapeDtypeStruct((NPAIR, PAIR), jnp.float32),
      ],
  )(tabT_t, tabT_l)


def _sc_gather_body(tab_t_hbm, tab_l_hbm, idx_t_hbm, idx_l_hbm,
                    out_t_hbm, out_l_hbm,
                    idx_t_v, idx_l_v, rows_v, sem):
  wid = lax.axis_index("s") * NC + lax.axis_index("c")
  base_chunk = wid * NCHUNK
  base_row = wid * B_PER_W

  pltpu.sync_copy(idx_t_hbm.at[pl.ds(base_chunk, NCHUNK)], idx_t_v)
  pltpu.sync_copy(idx_l_hbm.at[pl.ds(base_chunk, NCHUNK)], idx_l_v)

  copies = []
  for j in range(NCHUNK):
    copies.append(pltpu.async_copy(
        tab_t_hbm.at[idx_t_v.at[j]], rows_v.at[pl.ds(j * CHUNK, CHUNK)],
        sem))
  for c in copies:
    c.wait()
  pltpu.sync_copy(rows_v, out_t_hbm.at[pl.ds(base_row, B_PER_W)])

  copies = []
  for j in range(NCHUNK):
    copies.append(pltpu.async_copy(
        tab_l_hbm.at[idx_l_v.at[j]], rows_v.at[pl.ds(j * CHUNK, CHUNK)],
        sem))
  for c in copies:
    c.wait()
  pltpu.sync_copy(rows_v, out_l_hbm.at[pl.ds(base_row, B_PER_W)])


@functools.cache
def _sc_gather():
  return pl.kernel(
      _sc_gather_body,
      out_type=(
          jax.ShapeDtypeStruct((B, PAIR), jnp.float32),
          jax.ShapeDtypeStruct((B, PAIR), jnp.float32),
      ),
      mesh=plsc.VectorSubcoreMesh(core_axis_name="c", subcore_axis_name="s"),
      scratch_types=[
          pltpu.VMEM((NCHUNK, CHUNK), jnp.int32),
          pltpu.VMEM((NCHUNK, CHUNK), jnp.int32),
          pltpu.VMEM((B_PER_W, PAIR), jnp.float32),
          pltpu.SemaphoreType.DMA,
      ],
  )


BT = 2048


def _mlp_body(pt_ref, pl_ref, part_ref, parl_ref, ln1w_ref, ln1b_ref,
              w1t_ref, b1_ref, w2t_ref, b2_ref, ln2w_ref, ln2b_ref, out_ref):
  pt = pt_ref[...]
  plc = pl_ref[...]
  et = jnp.where(part_ref[...] > 0.5, pt[:, EMB:], pt[:, :EMB])
  el = jnp.where(parl_ref[...] > 0.5, plc[:, EMB:], plc[:, :EMB])
  x = jnp.concatenate([et, el], axis=1)
  mu = jnp.mean(x, axis=1, keepdims=True)
  xc = x - mu
  var = jnp.mean(xc * xc, axis=1, keepdims=True)
  h = xc * jax.lax.rsqrt(var + 1e-5) * ln1w_ref[...] + ln1b_ref[...]
  h = jnp.dot(h, w1t_ref[...], preferred_element_type=jnp.float32)
  h = jnp.maximum(h + b1_ref[...], 0.0)
  y = jnp.dot(h, w2t_ref[...], preferred_element_type=jnp.float32)
  y = y + b2_ref[...]
  mu2 = jnp.mean(y, axis=1, keepdims=True)
  yc = y - mu2
  var2 = jnp.mean(yc * yc, axis=1, keepdims=True)
  out_ref[...] = yc * jax.lax.rsqrt(var2 + 1e-5) * ln2w_ref[...] + ln2b_ref[...]


def _mlp(pt, plc, part, parl, ln1w, ln1b, w1t, b1, w2t, b2, ln2w, ln2b):
  full = lambda shape: pl.BlockSpec(shape, lambda i: (0, 0))
  return pl.pallas_call(
      _mlp_body,
      grid=(B // BT,),
      in_specs=[
          pl.BlockSpec((BT, PAIR), lambda i: (i, 0)),
          pl.BlockSpec((BT, PAIR), lambda i: (i, 0)),
          pl.BlockSpec((BT, 1), lambda i: (i, 0)),
          pl.BlockSpec((BT, 1), lambda i: (i, 0)),
          full((1, INPUT_DIM)), full((1, INPUT_DIM)),
          full((INPUT_DIM, HID)), full((1, HID)),
          full((HID, OUT)), full((1, OUT)),
          full((1, OUT)), full((1, OUT)),
      ],
      out_specs=pl.BlockSpec((BT, OUT), lambda i: (i, 0)),
      out_shape=jax.ShapeDtypeStruct((B, OUT), jnp.float32),
  )(pt, plc, part, parl, ln1w, ln1b, w1t, b1, w2t, b2, ln2w, ln2b)


def kernel(Type, Location, emb_type, emb_loc, ln1_w, ln1_b, w1, b1, w2, b2,
           ln2_w, ln2_b):
  it = Type.astype(jnp.int32)
  il = Location.astype(jnp.int32)
  pair = lambda r: ((r >> 11) << 10) | (r & 1023)
  half = lambda r: (r >> 10) & 1
  idx_t = jnp.reshape(pair(it), (B // CHUNK, CHUNK))
  idx_l = jnp.reshape(pair(il), (B // CHUNK, CHUNK))
  part = half(it).astype(jnp.float32).reshape(B, 1)
  parl = half(il).astype(jnp.float32).reshape(B, 1)
  ptab_t, ptab_l = _transpose(emb_type.T, emb_loc.T)
  pt, plc = _sc_gather()(ptab_t, ptab_l, idx_t, idx_l)
  return _mlp(
      pt, plc, part, parl,
      ln1_w.reshape(1, INPUT_DIM), ln1_b.reshape(1, INPUT_DIM),
      w1.T, b1.reshape(1, HID),
      w2.T, b2.reshape(1, OUT),
      ln2_w.reshape(1, OUT), ln2_b.reshape(1, OUT),
  )


# speedup vs baseline: 2.2622x; 1.3901x over previous
import functools

import jax
import jax.numpy as jnp
from jax import lax
from jax.experimental import pallas as pl
from jax.experimental.pallas import tpu as pltpu
from jax.experimental.pallas import tpu_sc as plsc

VOCAB = 1000000
B = 16384
EMB = 64
PAIR = 2 * EMB
INPUT_DIM = 2 * EMB
HID = 128
OUT = 64

NC = 2
NS = 16
NW = NC * NS
B_PER_W = B // NW
CHUNK = 128
NCHUNK = B_PER_W // CHUNK

TCOLS = 8192
HCOLS = TCOLS // 2
TGRID = (VOCAB + TCOLS - 1) // TCOLS
NPAIR = TGRID * HCOLS


def _pack_pairs(x, eye):
  xt = jax.lax.dot_general(x, eye, (((0,), (0,)), ((), ())),
                           preferred_element_type=jnp.float32)
  return jnp.concatenate([xt[:HCOLS], xt[HCOLS:]], axis=1)


def _transpose_body(at_ref, al_ref, ot_ref, ol_ref):
  eye = jnp.eye(EMB, dtype=jnp.float32)
  ot_ref[...] = _pack_pairs(at_ref[...], eye)
  ol_ref[...] = _pack_pairs(al_ref[...], eye)


def _transpose(tabT_t, tabT_l):
  return pl.pallas_call(
      _transpose_body,
      grid=(TGRID,),
      in_specs=[
          pl.BlockSpec((EMB, TCOLS), lambda i: (0, i)),
          pl.BlockSpec((EMB, TCOLS), lambda i: (0, i)),
      ],
      out_specs=[
          pl.BlockSpec((HCOLS, PAIR), lambda i: (i, 0)),
          pl.BlockSpec((HCOLS, PAIR), lambda i: (i, 0)),
      ],
      out_shape=[
          jax.ShapeDtypeStruct((NPAIR, PAIR), jnp.float32),
          jax.ShapeDtypeStruct((NPAIR, PAIR), jnp.float32),
      ],
  )(tabT_t, tabT_l)


def _sc_gather_body(tab_t_hbm, tab_l_hbm, idx_t_hbm, idx_l_hbm,
                    out_t_hbm, out_l_hbm,
                    idx_t_v, idx_l_v, rows_v, sem):
  wid = lax.axis_index("s") * NC + lax.axis_index("c")
  base_chunk = wid * NCHUNK
  base_row = wid * B_PER_W

  pltpu.sync_copy(idx_t_hbm.at[pl.ds(base_chunk, NCHUNK)], idx_t_v)
  pltpu.sync_copy(idx_l_hbm.at[pl.ds(base_chunk, NCHUNK)], idx_l_v)

  copies = []
  for j in range(NCHUNK):
    copies.append(pltpu.async_copy(
        tab_t_hbm.at[idx_t_v.at[j]], rows_v.at[pl.ds(j * CHUNK, CHUNK)],
        sem))
  for c in copies:
    c.wait()
  pltpu.sync_copy(rows_v, out_t_hbm.at[pl.ds(base_row, B_PER_W)])

  copies = []
  for j in range(NCHUNK):
    copies.append(pltpu.async_copy(
        tab_l_hbm.at[idx_l_v.at[j]], rows_v.at[pl.ds(j * CHUNK, CHUNK)],
        sem))
  for c in copies:
    c.wait()
  pltpu.sync_copy(rows_v, out_l_hbm.at[pl.ds(base_row, B_PER_W)])


@functools.cache
def _sc_gather():
  return pl.kernel(
      _sc_gather_body,
      out_type=(
          jax.ShapeDtypeStruct((B, PAIR), jnp.float32),
          jax.ShapeDtypeStruct((B, PAIR), jnp.float32),
      ),
      mesh=plsc.VectorSubcoreMesh(core_axis_name="c", subcore_axis_name="s"),
      scratch_types=[
          pltpu.VMEM((NCHUNK, CHUNK), jnp.int32),
          pltpu.VMEM((NCHUNK, CHUNK), jnp.int32),
          pltpu.VMEM((B_PER_W, PAIR), jnp.float32),
          pltpu.SemaphoreType.DMA,
      ],
  )


BT = 2048


def _mlp_body(pt_ref, pl_ref, part_ref, parl_ref, ln1w_ref, ln1b_ref,
              w1t_ref, b1_ref, w2t_ref, b2_ref, ln2w_ref, ln2b_ref, out_ref):
  pt = pt_ref[...]
  plc = pl_ref[...]
  et = jnp.where(part_ref[...] > 0.5, pt[:, EMB:], pt[:, :EMB])
  el = jnp.where(parl_ref[...] > 0.5, plc[:, EMB:], plc[:, :EMB])
  x = jnp.concatenate([et, el], axis=1)
  mu = jnp.mean(x, axis=1, keepdims=True)
  xc = x - mu
  var = jnp.mean(xc * xc, axis=1, keepdims=True)
  h = xc * jax.lax.rsqrt(var + 1e-5) * ln1w_ref[...] + ln1b_ref[...]
  h = jnp.dot(h, w1t_ref[...], preferred_element_type=jnp.float32)
  h = jnp.maximum(h + b1_ref[...], 0.0)
  y = jnp.dot(h, w2t_ref[...], preferred_element_type=jnp.float32)
  y = y + b2_ref[...]
  mu2 = jnp.mean(y, axis=1, keepdims=True)
  yc = y - mu2
  var2 = jnp.mean(yc * yc, axis=1, keepdims=True)
  out_ref[...] = yc * jax.lax.rsqrt(var2 + 1e-5) * ln2w_ref[...] + ln2b_ref[...]


def _mlp(pt, plc, part, parl, ln1w, ln1b, w1t, b1, w2t, b2, ln2w, ln2b):
  full = lambda shape: pl.BlockSpec(shape, lambda i: (0, 0))
  return pl.pallas_call(
      _mlp_body,
      grid=(B // BT,),
      in_specs=[
          pl.BlockSpec((BT, PAIR), lambda i: (i, 0)),
          pl.BlockSpec((BT, PAIR), lambda i: (i, 0)),
          pl.BlockSpec((BT, 1), lambda i: (i, 0)),
          pl.BlockSpec((BT, 1), lambda i: (i, 0)),
          full((1, INPUT_DIM)), full((1, INPUT_DIM)),
          full((INPUT_DIM, HID)), full((1, HID)),
          full((HID, OUT)), full((1, OUT)),
          full((1, OUT)), full((1, OUT)),
      ],
      out_specs=pl.BlockSpec((BT, OUT), lambda i: (i, 0)),
      out_shape=jax.ShapeDtypeStruct((B, OUT), jnp.float32),
  )(pt, plc, part, parl, ln1w, ln1b, w1t, b1, w2t, b2, ln2w, ln2b)


def kernel(Type, Location, emb_type, emb_loc, ln1_w, ln1_b, w1, b1, w2, b2,
           ln2_w, ln2_b):
  it = Type.astype(jnp.int32)
  il = Location.astype(jnp.int32)
  bs = TCOLS.bit_length() - 1
  pair = lambda r: ((r >> bs) << (bs - 1)) | (r & (HCOLS - 1))
  half = lambda r: (r >> (bs - 1)) & 1
  idx_t = jnp.reshape(pair(it), (B // CHUNK, CHUNK))
  idx_l = jnp.reshape(pair(il), (B // CHUNK, CHUNK))
  part = half(it).astype(jnp.float32).reshape(B, 1)
  parl = half(il).astype(jnp.float32).reshape(B, 1)
  ptab_t, ptab_l = _transpose(emb_type.T, emb_loc.T)
  pt, plc = _sc_gather()(ptab_t, ptab_l, idx_t, idx_l)
  return _mlp(
      pt, plc, part, parl,
      ln1_w.reshape(1, INPUT_DIM), ln1_b.reshape(1, INPUT_DIM),
      w1.T, b1.reshape(1, HID),
      w2.T, b2.reshape(1, OUT),
      ln2_w.reshape(1, OUT), ln2_b.reshape(1, OUT),
  )
